# Initial kernel scaffold; baseline (speedup 1.0000x reference)
import jax
import jax.numpy as jnp
from jax.experimental import pallas as pl


def kernel(input, target, _, labels, matric):
    raise NotImplementedError("write your pallas kernel here")


# speedup vs baseline: 29.6579x; 29.6579x over previous
import functools
import math

import jax
import jax.numpy as jnp
from jax import lax
from jax.experimental import pallas as pl
from jax.experimental.pallas import tpu as pltpu

_PAD_IDX = 0
_ALPHA = 0.1


def _tc_body(xt_ref, tgt_ref, pc_ref, m2_ref, num_ref, den_ref, *, smooth, V):
    i = pl.program_id(0)
    x = xt_ref[...]
    t = tgt_ref[0]
    pc = pc_ref[0]
    bt = x.shape[1]

    xmax = jnp.max(x, axis=0, keepdims=True)
    lse = jnp.log(jnp.sum(jnp.exp(x - xmax), axis=0, keepdims=True)) + xmax

    iota_v = lax.broadcasted_iota(jnp.int32, (V, bt), 0)
    onehot_c = (iota_v == t).astype(jnp.float32)
    x_c = jnp.sum(x * onehot_c, axis=0, keepdims=True)

    npair = m2_ref.shape[0]
    iota_p = lax.broadcasted_iota(jnp.int32, (npair, bt), 0)
    onehot_p = (iota_p == pc).astype(jnp.float32)
    wt = lax.dot_general(m2_ref[...], onehot_p,
                         dimension_numbers=(((0,), (0,)), ((), ())),
                         preferred_element_type=jnp.float32)

    mdotx = jnp.sum(wt * x, axis=0, keepdims=True)
    m_c = jnp.sum(wt * onehot_c, axis=0, keepdims=True)
    sum_m = jnp.sum(wt, axis=0, keepdims=True)

    ce = lse - x_c
    contrib = ce + smooth * (sum_m * x_c - m_c * ce - mdotx)
    valid = t != _PAD_IDX
    blk_num = jnp.sum(jnp.where(valid, contrib, 0.0))
    blk_den = jnp.sum(jnp.where(valid, 0.0, 1.0))

    @pl.when(i == 0)
    def _init():
        num_ref[0, 0] = 0.0
        den_ref[0, 0] = 0.0

    num_ref[0, 0] += blk_num
    den_ref[0, 0] += blk_den


def kernel(input, target, _, labels, matric):
    B, T, V = input.shape
    N = B * T
    BT = 2048
    nblk = N // BT

    import numpy as np
    length = np.float32(labels.shape[1] + 1.0)
    smooth = float(np.float32(1.0) - np.power(np.float32(1.0 - _ALPHA),
                                              np.float32(1.0) / length))

    xt = input.reshape(N, V).T
    tgt = target.reshape(N)
    prev = jnp.concatenate(
        [jnp.zeros((B, 1), dtype=target.dtype), target[:, :-1]], axis=1
    ).reshape(N)
    pc = prev * V + tgt
    tgt3 = tgt.reshape(nblk, 1, BT)
    pc3 = pc.reshape(nblk, 1, BT)
    m2 = matric[:-1, :-1, :-1].reshape(V * V, V)

    num, den = pl.pallas_call(
        functools.partial(_tc_body, smooth=smooth, V=V),
        grid=(nblk,),
        in_specs=[
            pl.BlockSpec((V, BT), lambda i: (0, i)),
            pl.BlockSpec((1, 1, BT), lambda i: (i, 0, 0)),
            pl.BlockSpec((1, 1, BT), lambda i: (i, 0, 0)),
            pl.BlockSpec((V * V, V), lambda i: (0, 0)),
        ],
        out_specs=[
            pl.BlockSpec(memory_space=pltpu.SMEM),
            pl.BlockSpec(memory_space=pltpu.SMEM),
        ],
        out_shape=[
            jax.ShapeDtypeStruct((1, 1), jnp.float32),
            jax.ShapeDtypeStruct((1, 1), jnp.float32),
        ],
    )(xt, tgt3, pc3, m2)
    return num[0, 0] / den[0, 0]
